# baseline (speedup 1.0000x reference)
import functools

import jax
import jax.numpy as jnp
from jax import lax
from jax.experimental import pallas as pl
from jax.experimental.pallas import tpu as pltpu
from jax.experimental.pallas import tpu_sc as plsc

L = 16384
D = 32
NC = 2
NS = 16
NW = NC * NS
BPW = L // NW

_MESH = plsc.VectorSubcoreMesh(core_axis_name="c", subcore_axis_name="s")
_NOTC = pltpu.CompilerParams(use_tc_tiling_on_sc=False)


def _band_body(bands, idx_hs, tbl_hs, out_h, ivs, rvs, sis, sgs, sws):
    wid = lax.axis_index("s") * NC + lax.axis_index("c")
    base = wid * BPW
    n = len(bands)
    ic = [pltpu.async_copy(idx_hs[k].at[pl.ds(base, BPW)], ivs[k], sis[k])
          for k in range(n)]
    gc = []
    for k in range(n):
        ic[k].wait()
        gc.append(pltpu.async_copy(tbl_hs[k].at[ivs[k]], rvs[k], sgs[k]))
    wc = []
    for k in range(n):
        gc[k].wait()
        wc.append(pltpu.async_copy(
            rvs[k], out_h.at[pl.ds(base, BPW), pl.ds(bands[k] * D, D)],
            sws[k]))
    for k in range(n):
        wc[k].wait()


def _make_band_kernel(bands):
    n = len(bands)
    scratch = (
        [pltpu.VMEM((BPW,), jnp.int32)] * n
        + [pltpu.VMEM((BPW, D), jnp.float32)] * n
        + [pltpu.SemaphoreType.DMA] * (3 * n)
    )

    @functools.partial(
        pl.kernel,
        mesh=_MESH,
        out_type=(),
        scratch_types=scratch,
        compiler_params=_NOTC,
        name=f"hgnn_bands_{'_'.join(map(str, bands))}",
    )
    def band_kernel(*args):
        idx_hs = args[:n]
        tbl_hs = args[n:2 * n]
        out_h = args[2 * n]
        rest = args[2 * n + 1:]
        ivs = rest[:n]
        rvs = rest[n:2 * n]
        sis = rest[2 * n:3 * n]
        sgs = rest[3 * n:4 * n]
        sws = rest[4 * n:5 * n]
        _band_body(bands, idx_hs, tbl_hs, out_h, ivs, rvs, sis, sgs, sws)

    return band_kernel


_k_small = _make_band_kernel((0, 2))
_k_big1 = _make_band_kernel((1,))
_k_big3 = _make_band_kernel((3,))


def kernel(dp, p, dl, l, Edp_emb, Eddp_emb, Edl_emb, Eddl_emb):
    dp = dp.astype(jnp.int32)
    p = p.astype(jnp.int32)
    dl = dl.astype(jnp.int32)
    l = l.astype(jnp.int32)
    out_ref = jax.new_ref(jnp.empty((L, 4 * D), jnp.float32))
    _k_big1(p, Eddp_emb, out_ref)
    _k_small(dp, dl, Edp_emb, Edl_emb, out_ref)
    _k_big3(l, Eddl_emb, out_ref)
    return out_ref[...].reshape(1, L, 4 * D)
